# baseline (speedup 1.0000x reference)
import functools

import jax
import jax.numpy as jnp
from jax import lax
from jax.experimental import pallas as pl
from jax.experimental.pallas import tpu as pltpu
from jax.experimental.pallas import tpu_sc as plsc

K = 512
T_BLK = 2048
N_TOK = 128 * 512
ROT = ((13, 15, 26, 6), (17, 29, 16, 24))


def _threefry_bits(x1, final_bias=0):
    k1 = jnp.uint32(0)
    k2 = jnp.uint32(42)
    ks = (k1, k2, k1 ^ k2 ^ jnp.uint32(0x1BD11BDA))
    assert final_bias in (0, 0x80000000)
    x0 = x1
    x1 = ((x1 << jnp.uint32(13)) | (x1 >> jnp.uint32(19))) ^ x0
    first = True
    for i in range(5):
        for r in ROT[i % 2]:
            if first:
                first = False
                continue
            x0 = x0 + x1
            x1 = ((x1 << jnp.uint32(r)) | (x1 >> jnp.uint32(32 - r))) ^ x0
        x0 = x0 + ks[(i + 1) % 3]
        extra = jnp.uint32(final_bias) if i == 4 else jnp.uint32(0)
        x1 = x1 + (ks[(i + 2) % 3] + jnp.uint32(i + 1) + extra)
    return x0 ^ x1


def _gumbel(sh):
    tiny = jnp.float32(1.1754943508222875e-38)
    u = jnp.where(sh == 0, tiny, sh.astype(jnp.float32) * jnp.float32(2.0 ** -23))
    return -jnp.log(-jnp.log(u))


CHUNK = 8


def _argmax_kernel(x_ref, comb_ref, sx_ref):
    g = pl.program_id(0)
    xs = x_ref[0]
    t_iota = lax.broadcasted_iota(jnp.uint32, (CHUNK, T_BLK), 1)
    p_iota = lax.broadcasted_iota(jnp.uint32, (CHUNK, T_BLK), 0)
    f0 = t_iota * jnp.uint32(K) + p_iota
    inv0 = jnp.uint32(K - 1) - p_iota
    base = jnp.uint32(g) * jnp.uint32(T_BLK * K)

    run = jnp.full((CHUNK, T_BLK), -(2 ** 31), jnp.int32)
    for c in range(K // CHUNK):
        x1 = f0 + (base + jnp.uint32(c * CHUNK + 42))
        bits_b = _threefry_bits(x1, final_bias=0x80000000)
        packed = ((bits_b & jnp.uint32(0xFFFFFE00))
                  | (inv0 - jnp.uint32(c * CHUNK)))
        run = jnp.maximum(run, packed.astype(jnp.int32))

    comb_ref[0] = jnp.max(run, axis=0, keepdims=True)

    x1x = (f0[0:1] + (base + jnp.uint32(42))) + xs.astype(jnp.uint32)
    sx_ref[0] = (_threefry_bits(x1x) >> jnp.uint32(9)).astype(jnp.int32)


def _combine_kernel(x_ref, comb_ref, sx_ref, diag_ref, off_ref, out_ref):
    xs = x_ref[:, 0, :]
    comb = comb_ref[:, 0, :].astype(jnp.uint32) ^ jnp.uint32(0x80000000)
    m_idx = (jnp.int32(K - 1) - (comb & jnp.uint32(0x1FF)).astype(jnp.int32))
    s_m = (comb >> jnp.uint32(9)).astype(jnp.int32)
    s_x = sx_ref[:, 0, :]
    log_diag = jnp.log(jnp.maximum(diag_ref[:, 0, :], jnp.float32(1e-12)))
    log_off = jnp.log(jnp.maximum(off_ref[:, 0, :], jnp.float32(1e-12)))
    a_other = _gumbel(s_m) + log_off
    a_self = _gumbel(s_x) + log_diag
    take_x = (a_self > a_other) | ((a_self == a_other) & (xs < m_idx))
    out_ref[:, 0, :] = jnp.where(take_x, xs, m_idx)


NW = 32
BPW = N_TOK // NW
_SC_MESH = plsc.VectorSubcoreMesh(core_axis_name="c", subcore_axis_name="s")


@functools.partial(
    pl.kernel,
    mesh=_SC_MESH,
    out_type=[
        jax.ShapeDtypeStruct((N_TOK,), jnp.float32),
        jax.ShapeDtypeStruct((N_TOK,), jnp.float32),
    ],
    scratch_types=[
        pltpu.VMEM((1024,), jnp.int32),
        pltpu.VMEM((1024,), jnp.float32),
        pltpu.VMEM((BPW,), jnp.int32),
        pltpu.VMEM((BPW,), jnp.float32),
        pltpu.VMEM((BPW,), jnp.float32),
        pltpu.SemaphoreType.DMA,
    ],
    compiler_params=pltpu.CompilerParams(needs_layout_passes=False),
)
def _sc_gather(qflat_hbm, x_hbm, diag_hbm, off_hbm,
               tabidx, tab, xv, outd, outo, sem):
    wid = lax.axis_index("s") * 2 + lax.axis_index("c")
    for i in range(K // 16):
        jv = lax.iota(jnp.int32, 16) + jnp.int32(16 * i)
        tabidx[pl.ds(16 * i, 16)] = jv * jnp.int32(K + 1)
        tabidx[pl.ds(K + 16 * i, 16)] = jv * jnp.int32(K) + (jv ^ jnp.int32(1))
    copies = [
        pltpu.async_copy(
            qflat_hbm.at[tabidx.at[pl.ds(128 * cidx, 128)]],
            tab.at[pl.ds(128 * cidx, 128)],
            sem,
        )
        for cidx in range(1024 // 128)
    ]
    for cp in copies:
        cp.wait()
    pltpu.sync_copy(x_hbm.at[pl.ds(wid * BPW, BPW)], xv)

    def body(i, _):
        sl = pl.ds(pl.multiple_of(i * 16, 16), 16)
        xi = xv[sl]
        outd[sl] = plsc.load_gather(tab, [xi])
        outo[sl] = plsc.load_gather(tab, [xi + jnp.int32(K)])
        return 0

    lax.fori_loop(0, BPW // 16, body, 0)
    pltpu.sync_copy(outd, diag_hbm.at[pl.ds(wid * BPW, BPW)])
    pltpu.sync_copy(outo, off_hbm.at[pl.ds(wid * BPW, BPW)])


def kernel(x, t, qtcum):
    qflat = lax.dynamic_slice(
        qtcum, (t, 0, 0), (1, K, K)).reshape(K * K)
    x3 = x.astype(jnp.int32).reshape(32, 1, T_BLK)

    diag_x, off_x = _sc_gather(qflat, x3.reshape(N_TOK))

    comb, s_x = pl.pallas_call(
        _argmax_kernel,
        grid=(32,),
        in_specs=[pl.BlockSpec((1, 1, T_BLK), lambda g: (g, 0, 0))],
        out_specs=[pl.BlockSpec((1, 1, T_BLK), lambda g: (g, 0, 0))] * 2,
        out_shape=[jax.ShapeDtypeStruct((32, 1, T_BLK), jnp.int32)] * 2,
    )(x3)

    out = pl.pallas_call(
        _combine_kernel,
        grid=(8,),
        in_specs=[pl.BlockSpec((4, 1, T_BLK), lambda g: (g, 0, 0))] * 5,
        out_specs=pl.BlockSpec((4, 1, T_BLK), lambda g: (g, 0, 0)),
        out_shape=jax.ShapeDtypeStruct((32, 1, T_BLK), jnp.int32),
    )(x3, comb, s_x,
      diag_x.reshape(32, 1, T_BLK), off_x.reshape(32, 1, T_BLK))
    return out.reshape(128, 512)


# speedup vs baseline: 1.0048x; 1.0048x over previous
import functools

import jax
import jax.numpy as jnp
from jax import lax
from jax.experimental import pallas as pl
from jax.experimental.pallas import tpu as pltpu
from jax.experimental.pallas import tpu_sc as plsc

K = 512
T_BLK = 1024
N_TOK = 128 * 512
ROT = ((13, 15, 26, 6), (17, 29, 16, 24))


def _threefry_bits(x1, final_bias=0):
    k1 = jnp.uint32(0)
    k2 = jnp.uint32(42)
    ks = (k1, k2, k1 ^ k2 ^ jnp.uint32(0x1BD11BDA))
    assert final_bias in (0, 0x80000000)
    x0 = x1
    x1 = ((x1 << jnp.uint32(13)) | (x1 >> jnp.uint32(19))) ^ x0
    first = True
    for i in range(5):
        for r in ROT[i % 2]:
            if first:
                first = False
                continue
            x0 = x0 + x1
            x1 = ((x1 << jnp.uint32(r)) | (x1 >> jnp.uint32(32 - r))) ^ x0
        x0 = x0 + ks[(i + 1) % 3]
        extra = jnp.uint32(final_bias) if i == 4 else jnp.uint32(0)
        x1 = x1 + (ks[(i + 2) % 3] + jnp.uint32(i + 1) + extra)
    return x0 ^ x1


def _gumbel(sh):
    tiny = jnp.float32(1.1754943508222875e-38)
    u = jnp.where(sh == 0, tiny, sh.astype(jnp.float32) * jnp.float32(2.0 ** -23))
    return -jnp.log(-jnp.log(u))


CHUNK = 8


def _argmax_kernel(x_ref, comb_ref, sx_ref):
    g = pl.program_id(0)
    xs = x_ref[0]
    t_iota = lax.broadcasted_iota(jnp.uint32, (CHUNK, T_BLK), 1)
    p_iota = lax.broadcasted_iota(jnp.uint32, (CHUNK, T_BLK), 0)
    f0 = t_iota * jnp.uint32(K) + p_iota
    inv0 = jnp.uint32(K - 1) - p_iota
    base = jnp.uint32(g) * jnp.uint32(T_BLK * K)

    run = jnp.full((CHUNK, T_BLK), -(2 ** 31), jnp.int32)
    for c in range(K // CHUNK):
        x1 = f0 + (base + jnp.uint32(c * CHUNK + 42))
        bits_b = _threefry_bits(x1, final_bias=0x80000000)
        packed = ((bits_b & jnp.uint32(0xFFFFFE00))
                  | (inv0 - jnp.uint32(c * CHUNK)))
        run = jnp.maximum(run, packed.astype(jnp.int32))

    comb_ref[0] = jnp.max(run, axis=0, keepdims=True)

    x1x = (f0[0:1] + (base + jnp.uint32(42))) + xs.astype(jnp.uint32)
    sx_ref[0] = (_threefry_bits(x1x) >> jnp.uint32(9)).astype(jnp.int32)


def _combine_kernel(x_ref, comb_ref, sx_ref, diag_ref, off_ref, out_ref):
    xs = x_ref[:, 0, :]
    comb = comb_ref[:, 0, :].astype(jnp.uint32) ^ jnp.uint32(0x80000000)
    m_idx = (jnp.int32(K - 1) - (comb & jnp.uint32(0x1FF)).astype(jnp.int32))
    s_m = (comb >> jnp.uint32(9)).astype(jnp.int32)
    s_x = sx_ref[:, 0, :]
    log_diag = jnp.log(jnp.maximum(diag_ref[:, 0, :], jnp.float32(1e-12)))
    log_off = jnp.log(jnp.maximum(off_ref[:, 0, :], jnp.float32(1e-12)))
    a_other = _gumbel(s_m) + log_off
    a_self = _gumbel(s_x) + log_diag
    take_x = (a_self > a_other) | ((a_self == a_other) & (xs < m_idx))
    out_ref[:, 0, :] = jnp.where(take_x, xs, m_idx)


NW = 32
BPW = N_TOK // NW
_SC_MESH = plsc.VectorSubcoreMesh(core_axis_name="c", subcore_axis_name="s")


@functools.partial(
    pl.kernel,
    mesh=_SC_MESH,
    out_type=[
        jax.ShapeDtypeStruct((N_TOK,), jnp.float32),
        jax.ShapeDtypeStruct((N_TOK,), jnp.float32),
    ],
    scratch_types=[
        pltpu.VMEM((1024,), jnp.int32),
        pltpu.VMEM((1024,), jnp.float32),
        pltpu.VMEM((BPW,), jnp.int32),
        pltpu.VMEM((BPW,), jnp.float32),
        pltpu.VMEM((BPW,), jnp.float32),
        pltpu.SemaphoreType.DMA,
    ],
    compiler_params=pltpu.CompilerParams(needs_layout_passes=False),
)
def _sc_gather(qflat_hbm, x_hbm, diag_hbm, off_hbm,
               tabidx, tab, xv, outd, outo, sem):
    wid = lax.axis_index("s") * 2 + lax.axis_index("c")
    for i in range(K // 16):
        jv = lax.iota(jnp.int32, 16) + jnp.int32(16 * i)
        tabidx[pl.ds(16 * i, 16)] = jv * jnp.int32(K + 1)
        tabidx[pl.ds(K + 16 * i, 16)] = jv * jnp.int32(K) + (jv ^ jnp.int32(1))
    copies = [
        pltpu.async_copy(
            qflat_hbm.at[tabidx.at[pl.ds(128 * cidx, 128)]],
            tab.at[pl.ds(128 * cidx, 128)],
            sem,
        )
        for cidx in range(1024 // 128)
    ]
    for cp in copies:
        cp.wait()
    pltpu.sync_copy(x_hbm.at[pl.ds(wid * BPW, BPW)], xv)

    def body(i, _):
        sl = pl.ds(pl.multiple_of(i * 16, 16), 16)
        xi = xv[sl]
        outd[sl] = plsc.load_gather(tab, [xi])
        outo[sl] = plsc.load_gather(tab, [xi + jnp.int32(K)])
        return 0

    lax.fori_loop(0, BPW // 16, body, 0)
    pltpu.sync_copy(outd, diag_hbm.at[pl.ds(wid * BPW, BPW)])
    pltpu.sync_copy(outo, off_hbm.at[pl.ds(wid * BPW, BPW)])


def kernel(x, t, qtcum):
    qflat = lax.dynamic_slice(
        qtcum, (t, 0, 0), (1, K, K)).reshape(K * K)
    x3 = x.astype(jnp.int32).reshape(64, 1, T_BLK)

    diag_x, off_x = _sc_gather(qflat, x3.reshape(N_TOK))

    comb, s_x = pl.pallas_call(
        _argmax_kernel,
        grid=(64,),
        in_specs=[pl.BlockSpec((1, 1, T_BLK), lambda g: (g, 0, 0))],
        out_specs=[pl.BlockSpec((1, 1, T_BLK), lambda g: (g, 0, 0))] * 2,
        out_shape=[jax.ShapeDtypeStruct((64, 1, T_BLK), jnp.int32)] * 2,
    )(x3)

    out = pl.pallas_call(
        _combine_kernel,
        grid=(8,),
        in_specs=[pl.BlockSpec((8, 1, T_BLK), lambda g: (g, 0, 0))] * 5,
        out_specs=pl.BlockSpec((8, 1, T_BLK), lambda g: (g, 0, 0)),
        out_shape=jax.ShapeDtypeStruct((64, 1, T_BLK), jnp.int32),
    )(x3, comb, s_x,
      diag_x.reshape(64, 1, T_BLK), off_x.reshape(64, 1, T_BLK))
    return out.reshape(128, 512)
